# baseline (speedup 1.0000x reference)
import functools
import jax
import jax.numpy as jnp
from jax import lax
from jax.experimental import pallas as pl
from jax.experimental.pallas import tpu as pltpu
from jax.experimental.pallas import tpu_sc as plsc

_SRC_ROWS = 16


def kernel(modified_adj):
    n = modified_adj.shape[0]
    info = plsc.get_sparse_core_info()
    nw = info.num_cores * info.num_subcores
    rows_per_w = n // nw
    iters = rows_per_w // _SRC_ROWS
    mesh = plsc.VectorSubcoreMesh(core_axis_name="c", subcore_axis_name="s")

    @functools.partial(
        pl.kernel,
        out_type=jax.ShapeDtypeStruct((n, n), jnp.float32),
        mesh=mesh,
        scratch_types=[
            pltpu.VMEM((_SRC_ROWS, n), jnp.float32),
            pltpu.SemaphoreType.DMA,
        ],
    )
    def sc_ones(src_hbm, out_hbm, buf_v, sem):
        wid = lax.axis_index("s") * info.num_cores + lax.axis_index("c")
        base = wid * rows_per_w
        pltpu.sync_copy(src_hbm, buf_v)

        def body(k, carry):
            pltpu.async_copy(
                buf_v, out_hbm.at[pl.ds(base + k * _SRC_ROWS, _SRC_ROWS), :],
                sem)
            return carry

        lax.fori_loop(0, iters, body, 0)

        def drain(k, carry):
            pltpu.make_async_copy(
                buf_v, out_hbm.at[pl.ds(base + k * _SRC_ROWS, _SRC_ROWS), :],
                sem).wait()
            return carry

        lax.fori_loop(0, iters, drain, 0)

    src = jnp.ones((_SRC_ROWS, n), jnp.float32)
    return sc_ones(src)
